# baseline (speedup 1.0000x reference)
import jax
import jax.numpy as jnp
from jax import lax
from jax.experimental import pallas as pl
from jax.experimental.pallas import tpu as pltpu
from jax.experimental.pallas import tpu_sc as plsc

N_NODES = 10000
N_EDGES = 160000
IN_SIZE = 256
OUT_SIZE = 256

NC = 2
NS = 16
HALF = OUT_SIZE // NC
CHUNK = 80
EPT = 10240
E_PAD = EPT * NS
NQ = EPT // CHUNK
N_PAD = 10240
ROWS_PER_SUB = N_PAD // NS

BN = 1000


def _matmul_block(x_ref, w_ref, o_ref):
    o_ref[0, :, :] = jnp.dot(x_ref[...], w_ref[...],
                             preferred_element_type=jnp.float32)


def _support_halves(x, weight):
    return pl.pallas_call(
        _matmul_block,
        grid=(NC, N_NODES // BN),
        in_specs=[
            pl.BlockSpec((BN, IN_SIZE), lambda c, i: (i, 0)),
            pl.BlockSpec((IN_SIZE, HALF), lambda c, i: (0, c)),
        ],
        out_specs=pl.BlockSpec((1, BN, HALF), lambda c, i: (c, i, 0)),
        out_shape=jax.ShapeDtypeStruct((NC, N_NODES, HALF), jnp.float32),
    )(x, weight)


def _sc_body(sup_ref, pk_ref, adj_ref, out_ref, idxb, adjb, gbufs, sbufs, acc,
             isems, gsems, ssems):
    c = lax.axis_index("c")
    s = lax.axis_index("s")
    sup = sup_ref.at[c]
    hbm_dummy = sup.at[pl.ds(0, CHUNK)]
    pk = pk_ref.at[s]
    pk_dummy = pk.at[0]
    adjh = adj_ref.at[s]
    adj_dummy = adjh.at[0]

    def _issue_idx(q, k4):
        pltpu.async_copy(pk.at[q], idxb[k4], isems[k4])
        pltpu.async_copy(adjh.at[q], adjb[k4], isems[k4])

    def _wait_idx(k4):
        pltpu.make_async_copy(pk_dummy, idxb[k4], isems[k4]).wait()
        pltpu.make_async_copy(adj_dummy, adjb[k4], isems[k4]).wait()

    def _zero_row(i, carry):
        for j in range(HALF // 16):
            sbufs[0][i, pl.ds(j * 16, 16)] = jnp.zeros((16,), jnp.float32)
        return carry
    lax.fori_loop(0, CHUNK, _zero_row, None)
    for k in range(ROWS_PER_SUB // CHUNK):
        pltpu.sync_copy(sbufs[0],
                        acc.at[pl.ds(s * ROWS_PER_SUB + k * CHUNK, CHUNK)])

    for q in range(4):
        _issue_idx(q, q)
    for q in range(2):
        _wait_idx(q)
        pltpu.async_copy(sup.at[idxb[q].at[0]], gbufs[q], gsems[q])
    plsc.subcore_barrier()

    def _visit(t, b4):
        q = 4 * t + b4
        b2 = b4 % 2
        k4 = (b4 + 2) % 4
        gbuf, sbuf = gbufs[b2], sbufs[b2]
        pltpu.make_async_copy(hbm_dummy, gbuf, gsems[b2]).wait()

        @pl.when(q >= 2)
        def _drain():
            pltpu.make_async_copy(hbm_dummy, sbuf, ssems[b2]).wait()

        @pl.when((q >= 2) & (q + 2 < NQ))
        def _refill_idx():
            _issue_idx(q + 2, k4)

        def _scale(g, carry):
            av = adjb[b4][pl.ds(g * 16, 16)]
            for l in range(16):
                a = av[l]
                e = g * 16 + l
                for j in range(HALF // 16):
                    sl = pl.ds(j * 16, 16)
                    sbuf[e, sl] = gbuf[e, sl] * a
            return carry
        lax.fori_loop(0, CHUNK // 16, _scale, None)

        pltpu.async_copy(sbuf, acc.at[idxb[b4].at[1]], ssems[b2], add=True)

        @pl.when(q + 2 < NQ)
        def _refill_gather():
            _wait_idx(k4)
            pltpu.async_copy(sup.at[idxb[k4].at[0]], gbuf, gsems[b2])

    def _step(t, carry):
        for b4 in range(4):
            _visit(t, b4)
        return carry
    lax.fori_loop(0, NQ // 4, _step, None)

    pltpu.make_async_copy(hbm_dummy, sbufs[0], ssems[0]).wait()
    pltpu.make_async_copy(hbm_dummy, sbufs[1], ssems[1]).wait()
    plsc.subcore_barrier()
    pltpu.sync_copy(acc.at[pl.ds(s * ROWS_PER_SUB, ROWS_PER_SUB)],
                    out_ref.at[c].at[pl.ds(s * ROWS_PER_SUB, ROWS_PER_SUB)])


def _sc_scatter(support_t, packed, adj_r):
    mesh = plsc.VectorSubcoreMesh(core_axis_name="c", subcore_axis_name="s")
    k = pl.kernel(
        lambda sup, pk, adj, out, *scr: _sc_body(
            sup, pk, adj, out,
            [scr[0], scr[1], scr[2], scr[3]],
            [scr[4], scr[5], scr[6], scr[7]],
            [scr[8], scr[9]],
            [scr[10], scr[11]],
            scr[12],
            [scr[13], scr[14], scr[15], scr[16]],
            [scr[17], scr[18]],
            [scr[19], scr[20]],
        ),
        out_type=jax.ShapeDtypeStruct((NC, N_PAD, HALF), jnp.float32),
        mesh=mesh,
        scratch_types=(
            [pltpu.VMEM((2, CHUNK), jnp.int32) for _ in range(4)]
            + [pltpu.VMEM((CHUNK,), jnp.float32) for _ in range(4)]
            + [pltpu.VMEM((CHUNK, HALF), jnp.float32) for _ in range(4)]
            + [pltpu.VMEM_SHARED((N_PAD, HALF), jnp.float32)]
            + [pltpu.SemaphoreType.DMA for _ in range(8)]
        ),
    )
    return k(support_t, packed, adj_r)


def kernel(x, edge_index, adj_values, weight):
    ei = edge_index.astype(jnp.int32)
    row_p = jnp.zeros((E_PAD,), jnp.int32).at[:N_EDGES].set(ei[0])
    col_p = jnp.zeros((E_PAD,), jnp.int32).at[:N_EDGES].set(ei[1])
    adj_p = jnp.zeros((E_PAD,), jnp.float32).at[:N_EDGES].set(adj_values)
    packed = jnp.stack([col_p.reshape(NS, NQ, CHUNK),
                        row_p.reshape(NS, NQ, CHUNK)], axis=2)
    support_t = _support_halves(x, weight)
    out2 = _sc_scatter(support_t, packed, adj_p.reshape(NS, NQ, CHUNK))
    return out2[:, :N_NODES, :].transpose(1, 0, 2).reshape(N_NODES, OUT_SIZE)


# speedup vs baseline: 1.0088x; 1.0088x over previous
import functools

import jax
import jax.numpy as jnp
from jax import lax
from jax.experimental import pallas as pl
from jax.experimental.pallas import tpu as pltpu
from jax.experimental.pallas import tpu_sc as plsc

N_NODES = 10000
N_EDGES = 160000
IN_SIZE = 256
OUT_SIZE = 256

NC = 2
NS = 16
HALF = OUT_SIZE // NC
CHUNK = 64
EPT = 10240
E_PAD = EPT * NS
NQ = EPT // CHUNK
N_PAD = 10240
ROWS_PER_SUB = N_PAD // NS

BN = 1000


def _matmul_block(x_ref, w_ref, o_ref):
    o_ref[0, :, :] = jnp.dot(x_ref[...], w_ref[...],
                             preferred_element_type=jnp.float32)


def _support_halves(x, weight):
    return pl.pallas_call(
        _matmul_block,
        grid=(NC, N_NODES // BN),
        in_specs=[
            pl.BlockSpec((BN, IN_SIZE), lambda c, i: (i, 0)),
            pl.BlockSpec((IN_SIZE, HALF), lambda c, i: (0, c)),
        ],
        out_specs=pl.BlockSpec((1, BN, HALF), lambda c, i: (c, i, 0)),
        out_shape=jax.ShapeDtypeStruct((NC, N_NODES, HALF), jnp.float32),
    )(x, weight)


def _sc_body(sup_ref, col_ref, row_ref, adj_ref, out_ref,
             colb, rowb, adjb, gbufs, sbufs, acc, isems, gsems, ssems):
    c = lax.axis_index("c")
    s = lax.axis_index("s")
    sup = sup_ref.at[c]
    hbm_dummy = sup.at[pl.ds(0, CHUNK)]
    ebase = s * EPT

    def _issue_idx(q, k4):
        sl = pl.ds(ebase + q * CHUNK, CHUNK)
        pltpu.async_copy(col_ref.at[sl], colb[k4], isems[k4])
        pltpu.async_copy(row_ref.at[sl], rowb[k4], isems[k4])
        pltpu.async_copy(adj_ref.at[sl], adjb[k4], isems[k4])

    idummy = pl.ds(0, CHUNK)

    def _wait_idx(k4):
        pltpu.make_async_copy(col_ref.at[idummy], colb[k4], isems[k4]).wait()
        pltpu.make_async_copy(row_ref.at[idummy], rowb[k4], isems[k4]).wait()
        pltpu.make_async_copy(adj_ref.at[idummy], adjb[k4], isems[k4]).wait()

    def _zero_row(i, carry):
        for j in range(HALF // 16):
            sbufs[0][i, pl.ds(j * 16, 16)] = jnp.zeros((16,), jnp.float32)
        return carry
    lax.fori_loop(0, CHUNK, _zero_row, None)
    for k in range(ROWS_PER_SUB // CHUNK):
        pltpu.sync_copy(sbufs[0],
                        acc.at[pl.ds(s * ROWS_PER_SUB + k * CHUNK, CHUNK)])

    for q in range(4):
        _issue_idx(q, q)
    for q in range(2):
        _wait_idx(q)
        pltpu.async_copy(sup.at[colb[q]], gbufs[q], gsems[q])
    plsc.subcore_barrier()

    def _visit(t, b4):
        q = 4 * t + b4
        b2 = b4 % 2
        k4 = (b4 + 2) % 4
        gbuf, sbuf = gbufs[b2], sbufs[b2]
        pltpu.make_async_copy(hbm_dummy, gbuf, gsems[b2]).wait()

        @pl.when(q >= 2)
        def _drain():
            pltpu.make_async_copy(hbm_dummy, sbuf, ssems[b2]).wait()

        @pl.when((q >= 2) & (q + 2 < NQ))
        def _refill_idx():
            _issue_idx(q + 2, k4)

        def _scale(g, carry):
            av = adjb[b4][pl.ds(g * 16, 16)]
            for l in range(16):
                a = av[l]
                e = g * 16 + l
                for j in range(HALF // 16):
                    sl = pl.ds(j * 16, 16)
                    sbuf[e, sl] = gbuf[e, sl] * a
            return carry
        lax.fori_loop(0, CHUNK // 16, _scale, None)

        pltpu.async_copy(sbuf, acc.at[rowb[b4]], ssems[b2], add=True)

        @pl.when(q + 2 < NQ)
        def _refill_gather():
            _wait_idx(k4)
            pltpu.async_copy(sup.at[colb[k4]], gbuf, gsems[b2])

    def _step(t, carry):
        for b4 in range(4):
            _visit(t, b4)
        return carry
    lax.fori_loop(0, NQ // 4, _step, None)

    pltpu.make_async_copy(hbm_dummy, sbufs[0], ssems[0]).wait()
    pltpu.make_async_copy(hbm_dummy, sbufs[1], ssems[1]).wait()
    plsc.subcore_barrier()
    pltpu.sync_copy(acc.at[pl.ds(s * ROWS_PER_SUB, ROWS_PER_SUB)],
                    out_ref.at[c].at[pl.ds(s * ROWS_PER_SUB, ROWS_PER_SUB)])


def _sc_scatter(support_t, col_p, row_p, adj_p):
    mesh = plsc.VectorSubcoreMesh(core_axis_name="c", subcore_axis_name="s")
    k = pl.kernel(
        lambda sup, col, row, adj, out, *scr: _sc_body(
            sup, col, row, adj, out,
            [scr[0], scr[1], scr[2], scr[3]],
            [scr[4], scr[5], scr[6], scr[7]],
            [scr[8], scr[9], scr[10], scr[11]],
            [scr[12], scr[13]],
            [scr[14], scr[15]],
            scr[16],
            [scr[17], scr[18], scr[19], scr[20]],
            [scr[21], scr[22]],
            [scr[23], scr[24]],
        ),
        out_type=jax.ShapeDtypeStruct((NC, N_PAD, HALF), jnp.float32),
        mesh=mesh,
        scratch_types=(
            [pltpu.VMEM((CHUNK,), jnp.int32) for _ in range(4)]
            + [pltpu.VMEM((CHUNK,), jnp.int32) for _ in range(4)]
            + [pltpu.VMEM((CHUNK,), jnp.float32) for _ in range(4)]
            + [pltpu.VMEM((CHUNK, HALF), jnp.float32) for _ in range(4)]
            + [pltpu.VMEM_SHARED((N_PAD, HALF), jnp.float32)]
            + [pltpu.SemaphoreType.DMA for _ in range(8)]
        ),
    )
    return k(support_t, col_p, row_p, adj_p)


def kernel(x, edge_index, adj_values, weight):
    ei = edge_index.astype(jnp.int32)
    row_p = jnp.zeros((E_PAD,), jnp.int32).at[:N_EDGES].set(ei[0])
    col_p = jnp.zeros((E_PAD,), jnp.int32).at[:N_EDGES].set(ei[1])
    adj_p = jnp.zeros((E_PAD,), jnp.float32).at[:N_EDGES].set(adj_values)
    support_t = _support_halves(x, weight)
    out2 = _sc_scatter(support_t, col_p, row_p, adj_p)
    return out2[:, :N_NODES, :].transpose(1, 0, 2).reshape(N_NODES, OUT_SIZE)


# speedup vs baseline: 1.0825x; 1.0730x over previous
import jax
import jax.numpy as jnp
from jax import lax
from jax.experimental import pallas as pl
from jax.experimental.pallas import tpu as pltpu
from jax.experimental.pallas import tpu_sc as plsc

N_NODES = 10000
N_EDGES = 160000
IN_SIZE = 256
OUT_SIZE = 256

NC = 2
NS = 16
HALF = OUT_SIZE // NC
CHUNK = 32
EPT = 10240
E_PAD = EPT * NS
NQ = EPT // CHUNK
N_PAD = 10240
ROWS_PER_SUB = N_PAD // NS

BN = 1000


def _matmul_block(x_ref, w_ref, o_ref):
    o_ref[0, :, :] = jnp.dot(x_ref[...], w_ref[...],
                             preferred_element_type=jnp.float32)


def _support_halves(x, weight):
    return pl.pallas_call(
        _matmul_block,
        grid=(NC, N_NODES // BN),
        in_specs=[
            pl.BlockSpec((BN, IN_SIZE), lambda c, i: (i, 0)),
            pl.BlockSpec((IN_SIZE, HALF), lambda c, i: (0, c)),
        ],
        out_specs=pl.BlockSpec((1, BN, HALF), lambda c, i: (c, i, 0)),
        out_shape=jax.ShapeDtypeStruct((NC, N_NODES, HALF), jnp.float32),
    )(x, weight)


def _sc_body(sup_ref, col_ref, row_ref, adj_ref, out_ref,
             colb, rowb, adjb, gbufs, sbufs, acc, isems, gsems, ssems):
    c = lax.axis_index("c")
    s = lax.axis_index("s")
    sup = sup_ref.at[c]
    hbm_dummy = sup.at[pl.ds(0, CHUNK)]
    ebase = s * EPT
    idummy = pl.ds(0, CHUNK)

    def _issue_idx(q, k8):
        sl = pl.ds(ebase + q * CHUNK, CHUNK)
        pltpu.async_copy(col_ref.at[sl], colb[k8], isems[k8])
        pltpu.async_copy(row_ref.at[sl], rowb[k8], isems[k8])
        pltpu.async_copy(adj_ref.at[sl], adjb[k8], isems[k8])

    def _wait_idx(k8):
        pltpu.make_async_copy(col_ref.at[idummy], colb[k8], isems[k8]).wait()
        pltpu.make_async_copy(row_ref.at[idummy], rowb[k8], isems[k8]).wait()
        pltpu.make_async_copy(adj_ref.at[idummy], adjb[k8], isems[k8]).wait()

    def _zero_row(i, carry):
        for j in range(HALF // 16):
            sbufs[0][i, pl.ds(j * 16, 16)] = jnp.zeros((16,), jnp.float32)
        return carry
    lax.fori_loop(0, CHUNK, _zero_row, None)
    for k in range(ROWS_PER_SUB // CHUNK):
        pltpu.sync_copy(sbufs[0],
                        acc.at[pl.ds(s * ROWS_PER_SUB + k * CHUNK, CHUNK)])

    for q in range(4):
        _issue_idx(q, q)
    for q in range(3):
        _wait_idx(q)
        pltpu.async_copy(sup.at[colb[q]], gbufs[q], gsems[q])
    plsc.subcore_barrier()

    def _visit(t, b8):
        q = 8 * t + b8
        b4 = b8 % 4
        gbuf, sbuf = gbufs[b4], sbufs[b4]
        pltpu.make_async_copy(hbm_dummy, gbuf, gsems[b4]).wait()

        @pl.when(q >= 4)
        def _drain():
            pltpu.make_async_copy(hbm_dummy, sbuf, ssems[b4]).wait()

        @pl.when(q + 4 < NQ)
        def _refill_idx():
            _issue_idx(q + 4, (b8 + 4) % 8)

        def _scale(g, carry):
            av = adjb[b8][pl.ds(g * 16, 16)]
            for l in range(16):
                a = av[l]
                e = g * 16 + l
                for j in range(HALF // 16):
                    sl = pl.ds(j * 16, 16)
                    sbuf[e, sl] = gbuf[e, sl] * a
            return carry
        lax.fori_loop(0, CHUNK // 16, _scale, None)

        pltpu.async_copy(sbuf, acc.at[rowb[b8]], ssems[b4], add=True)

        @pl.when(q + 3 < NQ)
        def _refill_gather():
            k8 = (b8 + 3) % 8
            _wait_idx(k8)
            pltpu.async_copy(sup.at[colb[k8]], gbufs[(b4 + 3) % 4],
                             gsems[(b4 + 3) % 4])

    def _step(t, carry):
        for b8 in range(8):
            _visit(t, b8)
        return carry
    lax.fori_loop(0, NQ // 8, _step, None)

    for b in range(4):
        pltpu.make_async_copy(hbm_dummy, sbufs[b], ssems[b]).wait()
    plsc.subcore_barrier()
    pltpu.sync_copy(acc.at[pl.ds(s * ROWS_PER_SUB, ROWS_PER_SUB)],
                    out_ref.at[c].at[pl.ds(s * ROWS_PER_SUB, ROWS_PER_SUB)])


def _sc_scatter(support_t, col_p, row_p, adj_p):
    mesh = plsc.VectorSubcoreMesh(core_axis_name="c", subcore_axis_name="s")
    k = pl.kernel(
        lambda sup, col, row, adj, out, *scr: _sc_body(
            sup, col, row, adj, out,
            list(scr[0:8]),
            list(scr[8:16]),
            list(scr[16:24]),
            list(scr[24:28]),
            list(scr[28:32]),
            scr[32],
            list(scr[33:41]),
            list(scr[41:45]),
            list(scr[45:49]),
        ),
        out_type=jax.ShapeDtypeStruct((NC, N_PAD, HALF), jnp.float32),
        mesh=mesh,
        scratch_types=(
            [pltpu.VMEM((CHUNK,), jnp.int32) for _ in range(8)]
            + [pltpu.VMEM((CHUNK,), jnp.int32) for _ in range(8)]
            + [pltpu.VMEM((CHUNK,), jnp.float32) for _ in range(8)]
            + [pltpu.VMEM((CHUNK, HALF), jnp.float32) for _ in range(8)]
            + [pltpu.VMEM_SHARED((N_PAD, HALF), jnp.float32)]
            + [pltpu.SemaphoreType.DMA for _ in range(16)]
        ),
    )
    return k(support_t, col_p, row_p, adj_p)


def kernel(x, edge_index, adj_values, weight):
    ei = edge_index.astype(jnp.int32)
    row_p = jnp.zeros((E_PAD,), jnp.int32).at[:N_EDGES].set(ei[0])
    col_p = jnp.zeros((E_PAD,), jnp.int32).at[:N_EDGES].set(ei[1])
    adj_p = jnp.zeros((E_PAD,), jnp.float32).at[:N_EDGES].set(adj_values)
    support_t = _support_halves(x, weight)
    out2 = _sc_scatter(support_t, col_p, row_p, adj_p)
    return out2[:, :N_NODES, :].transpose(1, 0, 2).reshape(N_NODES, OUT_SIZE)
